# baseline (speedup 1.0000x reference)
import functools

import jax
import jax.numpy as jnp
from jax import lax
from jax.experimental import pallas as pl
from jax.experimental.pallas import tpu as pltpu
from jax.experimental.pallas import tpu_sc as plsc

NC = 2
NS = 16
NW = NC * NS

NB = 4096
T = 200
D = 64
V = 8193

BW = NB // NW
TC_ = 4
NCH = T // TC_


def _body(idx_hbm, table_hbm, out_hbm, table_sh, idx_v, buf, buf_t,
          gs0, gs1, ws0, ws1, isem):
    sid = lax.axis_index("s")
    wid = sid * NC + lax.axis_index("c")
    bcol = wid * BW
    gsem = (gs0, gs1)
    wsem = (ws0, ws1)

    @pl.when(sid == 0)
    def _():
        pltpu.sync_copy(table_hbm, table_sh)

    pltpu.sync_copy(idx_hbm.at[pl.ds(0, TC_), pl.ds(bcol, BW)], idx_v.at[0])
    plsc.subcore_barrier()

    rows_g = [lax.iota(jnp.int32, 16) + 16 * g for g in range(8)]

    def fire_gather(idx_ref, s):
        pltpu.async_copy(table_sh.at[idx_ref], buf.at[s], gsem[s])

    def drain_gather(idx_ref, s):
        pltpu.make_async_copy(table_sh.at[idx_ref], buf.at[s], gsem[s]).wait()

    def fire_write(t, s):
        for dt in range(8):
            pltpu.async_copy(
                buf_t.at[s].at[pl.ds(dt * 8, 8)],
                out_hbm.at[t].at[dt].at[wid], wsem[s])

    def wait_write(t, s):
        for dt in range(8):
            pltpu.make_async_copy(
                buf_t.at[s].at[pl.ds(dt * 8, 8)],
                out_hbm.at[t].at[dt].at[wid], wsem[s]).wait()

    def transpose(s):
        for d in range(D):
            col = jnp.full((16,), d, jnp.int32)
            for g in range(8):
                vals = plsc.load_gather(buf.at[s], [rows_g[g], col])
                buf_t[s, d, pl.ds(g * 16, 16)] = vals

    fire_gather(idx_v.at[0].at[0], 0)

    def chunk(c, carry):
        cb = lax.rem(c, 2)
        @pl.when(c < NCH - 1)
        def _():
            pltpu.async_copy(
                idx_hbm.at[pl.ds((c + 1) * TC_, TC_), pl.ds(bcol, BW)],
                idx_v.at[1 - cb], isem)

        for tt in range(TC_):
            t = c * TC_ + tt
            s = tt % 2
            o = 1 - s
            drain_gather(idx_v.at[cb].at[tt], s)
            if tt < TC_ - 1:
                fire_gather(idx_v.at[cb].at[tt + 1], o)
            else:
                @pl.when(c < NCH - 1)
                def _():
                    pltpu.make_async_copy(
                        idx_hbm.at[pl.ds((c + 1) * TC_, TC_),
                                   pl.ds(bcol, BW)],
                        idx_v.at[1 - cb], isem).wait()
                    fire_gather(idx_v.at[1 - cb].at[0], o)
            @pl.when(t >= 2)
            def _():
                wait_write(t - 2, s)
            transpose(s)
            fire_write(t, s)
        return carry

    lax.fori_loop(0, NCH, chunk, 0)

    wait_write(T - 2, 0)
    wait_write(T - 1, 1)


@functools.partial(jax.jit, static_argnums=())
def kernel(token_indices, position_embedding_matrix):
    idx_t = token_indices.astype(jnp.int32).T
    run = pl.kernel(
        _body,
        out_type=jax.ShapeDtypeStruct((T, 8, NW, 8, 128), jnp.float32),
        mesh=plsc.VectorSubcoreMesh(core_axis_name="c", subcore_axis_name="s"),
        scratch_types=[
            pltpu.VMEM_SHARED((V, D), jnp.float32),
            pltpu.VMEM((2, TC_, BW), jnp.int32),
            pltpu.VMEM((2, BW, D), jnp.float32),
            pltpu.VMEM((2, D, BW), jnp.float32),
            pltpu.SemaphoreType.DMA,
            pltpu.SemaphoreType.DMA,
            pltpu.SemaphoreType.DMA,
            pltpu.SemaphoreType.DMA,
            pltpu.SemaphoreType.DMA,
        ],
        compiler_params=pltpu.CompilerParams(
            use_tc_tiling_on_sc=False, needs_layout_passes=False),
    )
    out_phys = run(idx_t, position_embedding_matrix)
    return jnp.transpose(out_phys, (2, 4, 0, 1, 3)).reshape(NB, T, D)


# speedup vs baseline: 2.6439x; 2.6439x over previous
import functools

import jax
import jax.numpy as jnp
from jax import lax
from jax.experimental import pallas as pl
from jax.experimental.pallas import tpu as pltpu
from jax.experimental.pallas import tpu_sc as plsc

NC = 2
NS = 16
NW = NC * NS

NB = 4096
T = 200
D = 64
V = 8193

BW = NB // NW
TC_ = 4
NCH = T // TC_


def _body(idx_hbm, table_hbm, out_hbm, table_sh, idx_v, buf, buf_t,
          gs0, gs1, ws0, ws1, isem):
    sid = lax.axis_index("s")
    wid = sid * NC + lax.axis_index("c")
    bcol = wid * BW
    gsem = (gs0, gs1)
    wsem = (ws0, ws1)

    @pl.when(sid == 0)
    def _():
        pltpu.sync_copy(table_hbm, table_sh)

    pltpu.sync_copy(idx_hbm.at[pl.ds(0, TC_), pl.ds(bcol, BW)], idx_v.at[0])
    plsc.subcore_barrier()

    rows_g = [lax.iota(jnp.int32, 16) + 16 * g for g in range(8)]

    def fire_gather(idx_ref, s):
        pltpu.async_copy(table_sh.at[idx_ref], buf.at[s], gsem[s])

    def drain_gather(idx_ref, s):
        pltpu.make_async_copy(table_sh.at[idx_ref], buf.at[s], gsem[s]).wait()

    def fire_write(t, s):
        for dt in range(8):
            pltpu.async_copy(
                buf_t.at[s, pl.ds(dt * 8, 8), pl.ds(0, BW)],
                out_hbm.at[t].at[dt].at[wid], wsem[s])

    def wait_write(t, s):
        for dt in range(8):
            pltpu.make_async_copy(
                buf_t.at[s, pl.ds(dt * 8, 8), pl.ds(0, BW)],
                out_hbm.at[t].at[dt].at[wid], wsem[s]).wait()

    def transpose(s):
        for b in range(BW):
            bvec = jnp.full((16,), b, jnp.int32)
            for g in range(4):
                vals = buf[s, b, pl.ds(g * 16, 16)]
                plsc.store_scatter(buf_t.at[s], [rows_g[g], bvec], vals)

    fire_gather(idx_v.at[0].at[0], 0)

    def chunk(c, carry):
        cb = lax.rem(c, 2)
        @pl.when(c < NCH - 1)
        def _():
            pltpu.async_copy(
                idx_hbm.at[pl.ds((c + 1) * TC_, TC_), pl.ds(bcol, BW)],
                idx_v.at[1 - cb], isem)

        for tt in range(TC_):
            t = c * TC_ + tt
            s = tt % 2
            o = 1 - s
            drain_gather(idx_v.at[cb].at[tt], s)
            if tt < TC_ - 1:
                fire_gather(idx_v.at[cb].at[tt + 1], o)
            else:
                @pl.when(c < NCH - 1)
                def _():
                    pltpu.make_async_copy(
                        idx_hbm.at[pl.ds((c + 1) * TC_, TC_),
                                   pl.ds(bcol, BW)],
                        idx_v.at[1 - cb], isem).wait()
                    fire_gather(idx_v.at[1 - cb].at[0], o)
            @pl.when(t >= 2)
            def _():
                wait_write(t - 2, s)
            transpose(s)
            fire_write(t, s)
        return carry

    lax.fori_loop(0, NCH, chunk, 0)

    wait_write(T - 2, 0)
    wait_write(T - 1, 1)


@functools.partial(jax.jit, static_argnums=())
def kernel(token_indices, position_embedding_matrix):
    idx_t = token_indices.astype(jnp.int32).T
    run = pl.kernel(
        _body,
        out_type=jax.ShapeDtypeStruct((T, 8, NW, 8, 128), jnp.float32),
        mesh=plsc.VectorSubcoreMesh(core_axis_name="c", subcore_axis_name="s"),
        scratch_types=[
            pltpu.VMEM_SHARED((V, D), jnp.float32),
            pltpu.VMEM((2, TC_, BW), jnp.int32),
            pltpu.VMEM((2, BW, D), jnp.float32),
            pltpu.VMEM((2, D, BW + 1), jnp.float32),
            pltpu.SemaphoreType.DMA,
            pltpu.SemaphoreType.DMA,
            pltpu.SemaphoreType.DMA,
            pltpu.SemaphoreType.DMA,
            pltpu.SemaphoreType.DMA,
        ],
        compiler_params=pltpu.CompilerParams(
            use_tc_tiling_on_sc=False, needs_layout_passes=False),
    )
    out_phys = run(idx_t, position_embedding_matrix)
    return jnp.transpose(out_phys, (2, 4, 0, 1, 3)).reshape(NB, T, D)


# speedup vs baseline: 2.8244x; 1.0682x over previous
import functools

import jax
import jax.numpy as jnp
from jax import lax
from jax.experimental import pallas as pl
from jax.experimental.pallas import tpu as pltpu
from jax.experimental.pallas import tpu_sc as plsc

NC = 2
NS = 16
NW = NC * NS

NB = 4096
T = 200
D = 64
V = 8193

BW = NB // NW
TC_ = 4
NCH = T // TC_


def _body(idx_hbm, table_hbm, out_hbm, table_sh, idx_v, buf, buf_t,
          gs0, gs1, ws0, ws1, isem):
    sid = lax.axis_index("s")
    wid = sid * NC + lax.axis_index("c")
    bcol = wid * BW
    gsem = (gs0, gs1)
    wsem = (ws0, ws1)

    @pl.when(sid == 0)
    def _():
        pltpu.sync_copy(table_hbm, table_sh)

    pltpu.sync_copy(idx_hbm.at[pl.ds(0, TC_), pl.ds(bcol, BW)], idx_v.at[0])
    plsc.subcore_barrier()

    rows_g = [lax.iota(jnp.int32, 16) + 16 * g for g in range(8)]

    def fire_gather(idx_ref, s):
        pltpu.async_copy(table_sh.at[idx_ref], buf.at[s], gsem[s])

    def drain_gather(idx_ref, s):
        pltpu.make_async_copy(table_sh.at[idx_ref], buf.at[s], gsem[s]).wait()

    def fire_write(t, s):
        pltpu.async_copy(
            buf_t.at[s, :, :, pl.ds(0, BW)],
            out_hbm.at[t].at[:, wid], wsem[s])

    def wait_write(t, s):
        pltpu.make_async_copy(
            buf_t.at[s, :, :, pl.ds(0, BW)],
            out_hbm.at[t].at[:, wid], wsem[s]).wait()

    dt_g = [r // 8 for r in rows_g[:4]]
    d8_g = [lax.rem(r, 8) for r in rows_g[:4]]

    def transpose(s):
        for b in range(BW):
            bvec = jnp.full((16,), b, jnp.int32)
            for g in range(4):
                vals = buf[s, b, pl.ds(g * 16, 16)]
                plsc.store_scatter(
                    buf_t.at[s], [dt_g[g], d8_g[g], bvec], vals)

    fire_gather(idx_v.at[0].at[0], 0)

    def chunk(c, carry):
        cb = lax.rem(c, 2)
        @pl.when(c < NCH - 1)
        def _():
            pltpu.async_copy(
                idx_hbm.at[pl.ds((c + 1) * TC_, TC_), pl.ds(bcol, BW)],
                idx_v.at[1 - cb], isem)

        for tt in range(TC_):
            t = c * TC_ + tt
            s = tt % 2
            o = 1 - s
            drain_gather(idx_v.at[cb].at[tt], s)
            if tt < TC_ - 1:
                fire_gather(idx_v.at[cb].at[tt + 1], o)
            else:
                @pl.when(c < NCH - 1)
                def _():
                    pltpu.make_async_copy(
                        idx_hbm.at[pl.ds((c + 1) * TC_, TC_),
                                   pl.ds(bcol, BW)],
                        idx_v.at[1 - cb], isem).wait()
                    fire_gather(idx_v.at[1 - cb].at[0], o)
            @pl.when(t >= 2)
            def _():
                wait_write(t - 2, s)
            transpose(s)
            fire_write(t, s)
        return carry

    lax.fori_loop(0, NCH, chunk, 0)

    wait_write(T - 2, 0)
    wait_write(T - 1, 1)


@functools.partial(jax.jit, static_argnums=())
def kernel(token_indices, position_embedding_matrix):
    idx_t = token_indices.astype(jnp.int32).T
    run = pl.kernel(
        _body,
        out_type=jax.ShapeDtypeStruct((T, 8, NW, 8, 128), jnp.float32),
        mesh=plsc.VectorSubcoreMesh(core_axis_name="c", subcore_axis_name="s"),
        scratch_types=[
            pltpu.VMEM_SHARED((V, D), jnp.float32),
            pltpu.VMEM((2, TC_, BW), jnp.int32),
            pltpu.VMEM((2, BW, D), jnp.float32),
            pltpu.VMEM((2, 8, 8, BW + 1), jnp.float32),
            pltpu.SemaphoreType.DMA,
            pltpu.SemaphoreType.DMA,
            pltpu.SemaphoreType.DMA,
            pltpu.SemaphoreType.DMA,
            pltpu.SemaphoreType.DMA,
        ],
        compiler_params=pltpu.CompilerParams(
            use_tc_tiling_on_sc=False, needs_layout_passes=False),
    )
    out_phys = run(idx_t, position_embedding_matrix)
    return jnp.transpose(out_phys, (2, 4, 0, 1, 3)).reshape(NB, T, D)


# speedup vs baseline: 2.8463x; 1.0078x over previous
import functools

import jax
import jax.numpy as jnp
from jax import lax
from jax.experimental import pallas as pl
from jax.experimental.pallas import tpu as pltpu
from jax.experimental.pallas import tpu_sc as plsc

NC = 2
NS = 16
NW = NC * NS

NB = 4096
T = 200
D = 64
V = 8193

BW = NB // NW
TC_ = 4
NCH = T // TC_


def _body(idx_hbm, table_hbm, out_hbm, table_sh, idx_v, buf, buf_t,
          gs0, gs1, ws0, ws1, isem):
    sid = lax.axis_index("s")
    wid = sid * NC + lax.axis_index("c")
    bcol = wid * BW
    gsem = (gs0, gs1)
    wsem = (ws0, ws1)

    @pl.when(sid == 0)
    def _():
        pltpu.sync_copy(table_hbm, table_sh)

    pltpu.sync_copy(idx_hbm.at[pl.ds(0, TC_), pl.ds(bcol, BW)], idx_v.at[0])
    plsc.subcore_barrier()

    rows_g = [lax.iota(jnp.int32, 16) + 16 * g for g in range(8)]

    def fire_gather(idx_ref, s):
        pltpu.async_copy(table_sh.at[idx_ref], buf.at[s], gsem[s])

    def drain_gather(idx_ref, s):
        pltpu.make_async_copy(table_sh.at[idx_ref], buf.at[s], gsem[s]).wait()

    def fire_write(t, s):
        pltpu.async_copy(
            buf_t.at[s, :, :, pl.ds(0, BW)],
            out_hbm.at[t].at[:, wid], wsem[s])

    def wait_write(t, s):
        pltpu.make_async_copy(
            buf_t.at[s, :, :, pl.ds(0, BW)],
            out_hbm.at[t].at[:, wid], wsem[s]).wait()

    dt_g = [r // 8 for r in rows_g[:4]]
    d8_g = [lax.rem(r, 8) for r in rows_g[:4]]

    def transpose(s):
        for b in range(BW):
            bvec = jnp.full((16,), b, jnp.int32)
            for g in range(4):
                vals = buf[s, b, pl.ds(g * 16, 16)]
                plsc.store_scatter(
                    buf_t.at[s], [dt_g[g], d8_g[g], bvec], vals)

    fire_gather(idx_v.at[0].at[0], 0)

    def chunk(c, carry):
        cb = lax.rem(c, 2)
        @pl.when(c < NCH - 1)
        def _():
            pltpu.async_copy(
                idx_hbm.at[pl.ds((c + 1) * TC_, TC_), pl.ds(bcol, BW)],
                idx_v.at[1 - cb], isem)

        for tt in range(TC_):
            t = c * TC_ + tt
            s = tt % 2
            o = 1 - s
            drain_gather(idx_v.at[cb].at[tt], s)
            if tt < TC_ - 1:
                fire_gather(idx_v.at[cb].at[tt + 1], o)
            else:
                @pl.when(c < NCH - 1)
                def _():
                    pltpu.make_async_copy(
                        idx_hbm.at[pl.ds((c + 1) * TC_, TC_),
                                   pl.ds(bcol, BW)],
                        idx_v.at[1 - cb], isem).wait()
                    fire_gather(idx_v.at[1 - cb].at[0], o)
            @pl.when(t >= 2)
            def _():
                wait_write(t - 2, s)
            transpose(s)
            fire_write(t, s)
        return carry

    lax.fori_loop(0, NCH, chunk, 0)

    wait_write(T - 2, 0)
    wait_write(T - 1, 1)


@functools.partial(jax.jit, static_argnums=())
def kernel(token_indices, position_embedding_matrix):
    idx_t = token_indices.astype(jnp.int32).T
    run = pl.kernel(
        _body,
        out_type=jax.ShapeDtypeStruct((T, 8, NW, 8, 128), jnp.float32),
        mesh=plsc.VectorSubcoreMesh(core_axis_name="c", subcore_axis_name="s"),
        scratch_types=[
            pltpu.VMEM_SHARED((V, D), jnp.float32),
            pltpu.VMEM((2, TC_, BW), jnp.int32),
            pltpu.VMEM((2, BW, D), jnp.float32),
            pltpu.VMEM((2, 8, 8, BW + 8), jnp.float32),
            pltpu.SemaphoreType.DMA,
            pltpu.SemaphoreType.DMA,
            pltpu.SemaphoreType.DMA,
            pltpu.SemaphoreType.DMA,
            pltpu.SemaphoreType.DMA,
        ],
        compiler_params=pltpu.CompilerParams(
            use_tc_tiling_on_sc=False, needs_layout_passes=False),
    )
    out_phys = run(idx_t, position_embedding_matrix)
    return jnp.transpose(out_phys, (2, 4, 0, 1, 3)).reshape(NB, T, D)


# speedup vs baseline: 2.8518x; 1.0019x over previous
import functools

import jax
import jax.numpy as jnp
from jax import lax
from jax.experimental import pallas as pl
from jax.experimental.pallas import tpu as pltpu
from jax.experimental.pallas import tpu_sc as plsc

NC = 2
NS = 16
NW = NC * NS

NB = 4096
T = 200
D = 64
V = 8193

BW = NB // NW
TC_ = 4
NCH = T // TC_


def _body(idx_hbm, table_hbm, out_hbm, table_sh, idx_v, buf, buf_t,
          gs0, gs1, ws0, ws1, isem):
    sid = lax.axis_index("s")
    wid = sid * NC + lax.axis_index("c")
    bcol = wid * BW
    gsem = (gs0, gs1)
    wsem = (ws0, ws1)

    @pl.when(sid == 0)
    def _():
        pltpu.sync_copy(table_hbm, table_sh)

    pltpu.sync_copy(idx_hbm.at[pl.ds(0, TC_), pl.ds(bcol, BW)], idx_v.at[0])
    plsc.subcore_barrier()

    rows_g = [lax.iota(jnp.int32, 16) + 16 * g for g in range(8)]

    def fire_gather(idx_ref, s):
        pltpu.async_copy(table_sh.at[idx_ref], buf.at[s], gsem[s])

    def drain_gather(idx_ref, s):
        pltpu.make_async_copy(table_sh.at[idx_ref], buf.at[s], gsem[s]).wait()

    def fire_write(t, s):
        pltpu.async_copy(
            buf_t.at[s, :, :, pl.ds(0, BW)],
            out_hbm.at[t].at[:, wid], wsem[s])

    def wait_write(t, s):
        pltpu.make_async_copy(
            buf_t.at[s, :, :, pl.ds(0, BW)],
            out_hbm.at[t].at[:, wid], wsem[s]).wait()

    dt_g = [r // 8 for r in rows_g[:4]]
    d8_g = [lax.rem(r, 8) for r in rows_g[:4]]

    def transpose(s):
        for g in range(4):
            for b in range(BW):
                bvec = jnp.full((16,), b, jnp.int32)
                vals = buf[s, b, pl.ds(g * 16, 16)]
                plsc.store_scatter(
                    buf_t.at[s], [dt_g[g], d8_g[g], bvec], vals)

    fire_gather(idx_v.at[0].at[0], 0)

    def chunk(c, carry):
        cb = lax.rem(c, 2)
        @pl.when(c < NCH - 1)
        def _():
            pltpu.async_copy(
                idx_hbm.at[pl.ds((c + 1) * TC_, TC_), pl.ds(bcol, BW)],
                idx_v.at[1 - cb], isem)

        for tt in range(TC_):
            t = c * TC_ + tt
            s = tt % 2
            o = 1 - s
            drain_gather(idx_v.at[cb].at[tt], s)
            if tt < TC_ - 1:
                fire_gather(idx_v.at[cb].at[tt + 1], o)
            else:
                @pl.when(c < NCH - 1)
                def _():
                    pltpu.make_async_copy(
                        idx_hbm.at[pl.ds((c + 1) * TC_, TC_),
                                   pl.ds(bcol, BW)],
                        idx_v.at[1 - cb], isem).wait()
                    fire_gather(idx_v.at[1 - cb].at[0], o)
            @pl.when(t >= 2)
            def _():
                wait_write(t - 2, s)
            transpose(s)
            fire_write(t, s)
        return carry

    lax.fori_loop(0, NCH, chunk, 0)

    wait_write(T - 2, 0)
    wait_write(T - 1, 1)


@functools.partial(jax.jit, static_argnums=())
def kernel(token_indices, position_embedding_matrix):
    idx_t = token_indices.astype(jnp.int32).T
    run = pl.kernel(
        _body,
        out_type=jax.ShapeDtypeStruct((T, 8, NW, 8, 128), jnp.float32),
        mesh=plsc.VectorSubcoreMesh(core_axis_name="c", subcore_axis_name="s"),
        scratch_types=[
            pltpu.VMEM_SHARED((V, D), jnp.float32),
            pltpu.VMEM((2, TC_, BW), jnp.int32),
            pltpu.VMEM((2, BW, D), jnp.float32),
            pltpu.VMEM((2, 8, 8, BW + 8), jnp.float32),
            pltpu.SemaphoreType.DMA,
            pltpu.SemaphoreType.DMA,
            pltpu.SemaphoreType.DMA,
            pltpu.SemaphoreType.DMA,
            pltpu.SemaphoreType.DMA,
        ],
        compiler_params=pltpu.CompilerParams(
            use_tc_tiling_on_sc=False, needs_layout_passes=False),
    )
    out_phys = run(idx_t, position_embedding_matrix)
    return jnp.transpose(out_phys, (2, 4, 0, 1, 3)).reshape(NB, T, D)
